# Initial kernel scaffold; baseline (speedup 1.0000x reference)
import jax
import jax.numpy as jnp
from jax.experimental import pallas as pl


def kernel(x, labels, centers):
    raise NotImplementedError("write your pallas kernel here")


# speedup vs baseline: 2.7561x; 2.7561x over previous
import functools

import jax
import jax.numpy as jnp
from jax import lax
from jax.experimental import pallas as pl
from jax.experimental.pallas import tpu as pltpu
from jax.experimental.pallas import tpu_sc as plsc

BATCH = 4096
FEAT = 512
NCLS = 1000
NCORES = 2
NSUB = 16
NWORK = NCORES * NSUB
SPW = BATCH // NWORK
CHUNK = 32
NCHUNK = SPW // CHUNK
LANES = 16
ROW = 2 * FEAT


def _lane_allreduce_sum(v):
    lane = lax.iota(jnp.int32, LANES)
    dnums = lax.GatherDimensionNumbers(
        offset_dims=(), collapsed_slice_dims=(0,), start_index_map=(0,))
    for k in (8, 4, 2, 1):
        perm = lane ^ k
        v = v + lax.gather(v, perm[:, None], dnums, slice_sizes=(1,),
                           mode=lax.GatherScatterMode.PROMISE_IN_BOUNDS)
    return v


def _worker_body(x_hbm, labels_hbm, centers_hbm, out_hbm,
                 labels_v, x_v, c_v, out_v, sem_x, sem_c):
    wid = lax.axis_index("s") * NCORES + lax.axis_index("c")
    base = wid * SPW

    pltpu.sync_copy(labels_hbm.at[pl.ds(base, SPW)], labels_v)

    def chunk_acc(ci, vtot):
        cx = pltpu.async_copy(x_hbm.at[pl.ds(base + ci * CHUNK, CHUNK)],
                              x_v, sem_x)
        cc = pltpu.async_copy(centers_hbm.at[labels_v.at[pl.ds(ci * CHUNK, CHUNK)]],
                              c_v, sem_c)
        cx.wait()
        cc.wait()

        def sample_acc(s, vt):
            acc0 = jnp.zeros((LANES,), jnp.float32)
            acc1 = jnp.zeros((LANES,), jnp.float32)
            for j in range(FEAT // LANES):
                xv = x_v[s, pl.ds(j * LANES, LANES)]
                d0 = xv - c_v[s, pl.ds(j * LANES, LANES)]
                d1 = xv - c_v[s, pl.ds(FEAT + j * LANES, LANES)]
                acc0 = acc0 + d0 * d0
                acc1 = acc1 + d1 * d1
            take0 = _lane_allreduce_sum(acc0 - acc1) < 0.0
            sel = jnp.where(take0, acc0, acc1)
            return vt + sel

        return lax.fori_loop(0, CHUNK, sample_acc, vtot)

    vtot = jnp.zeros((LANES,), jnp.float32)
    for ci in range(NCHUNK):
        vtot = chunk_acc(ci, vtot)

    out_v[...] = vtot
    pltpu.sync_copy(out_v, out_hbm.at[wid])


@jax.jit
def kernel(x, labels, centers):
    centers_r = centers.reshape(NCLS, ROW)
    labels_i = labels.astype(jnp.int32)
    mesh = plsc.VectorSubcoreMesh(core_axis_name="c", subcore_axis_name="s")
    run = functools.partial(
        pl.kernel,
        mesh=mesh,
        out_type=jax.ShapeDtypeStruct((NWORK, LANES), jnp.float32),
        scratch_types=[
            pltpu.VMEM((SPW,), jnp.int32),
            pltpu.VMEM((CHUNK, FEAT), jnp.float32),
            pltpu.VMEM((CHUNK, ROW), jnp.float32),
            pltpu.VMEM((LANES,), jnp.float32),
            pltpu.SemaphoreType.DMA,
            pltpu.SemaphoreType.DMA,
        ],
    )(_worker_body)
    partials = run(x, labels_i, centers_r)
    return jnp.sum(partials) / BATCH


# speedup vs baseline: 3.1799x; 1.1538x over previous
import functools

import jax
import jax.numpy as jnp
from jax import lax
from jax.experimental import pallas as pl
from jax.experimental.pallas import tpu as pltpu
from jax.experimental.pallas import tpu_sc as plsc

BATCH = 4096
FEAT = 512
NCLS = 1000
NCORES = 2
NSUB = 16
NWORK = NCORES * NSUB
SPW = BATCH // NWORK
CHUNK = 16
NCHUNK = SPW // CHUNK
NBUF = 3
LANES = 16
ROW = 2 * FEAT


def _lane_allreduce_sum(v):
    lane = lax.iota(jnp.int32, LANES)
    dnums = lax.GatherDimensionNumbers(
        offset_dims=(), collapsed_slice_dims=(0,), start_index_map=(0,))
    for k in (8, 4, 2, 1):
        perm = lane ^ k
        v = v + lax.gather(v, perm[:, None], dnums, slice_sizes=(1,),
                           mode=lax.GatherScatterMode.PROMISE_IN_BOUNDS)
    return v


def _worker_body(x_hbm, labels_hbm, centers_hbm, out_hbm,
                 labels_v, x_v, c_v, out_v, sem_x, sem_c0, sem_c1, sem_c2):
    wid = lax.axis_index("s") * NCORES + lax.axis_index("c")
    base = wid * SPW
    sems = (sem_c0, sem_c1, sem_c2)

    cp_x = pltpu.async_copy(x_hbm.at[pl.ds(base, SPW)], x_v, sem_x)
    pltpu.sync_copy(labels_hbm.at[pl.ds(base, SPW)], labels_v)

    def issue(ci):
        b = ci % NBUF
        return pltpu.async_copy(
            centers_hbm.at[labels_v.at[pl.ds(ci * CHUNK, CHUNK)]],
            c_v.at[b], sems[b])

    cps = {}
    for ci in range(NBUF - 1):
        cps[ci] = issue(ci)
    cp_x.wait()

    def chunk_acc(ci, vtot):
        b = ci % NBUF

        def sample_acc(s, vt):
            acc0 = jnp.zeros((LANES,), jnp.float32)
            acc1 = jnp.zeros((LANES,), jnp.float32)
            for j in range(FEAT // LANES):
                xv = x_v[ci * CHUNK + s, pl.ds(j * LANES, LANES)]
                d0 = xv - c_v[b, s, pl.ds(j * LANES, LANES)]
                d1 = xv - c_v[b, s, pl.ds(FEAT + j * LANES, LANES)]
                acc0 = acc0 + d0 * d0
                acc1 = acc1 + d1 * d1
            take0 = _lane_allreduce_sum(acc0 - acc1) < 0.0
            sel = jnp.where(take0, acc0, acc1)
            return vt + sel

        return lax.fori_loop(0, CHUNK, sample_acc, vtot)

    vtot = jnp.zeros((LANES,), jnp.float32)
    for ci in range(NCHUNK):
        nxt = ci + NBUF - 1
        if nxt < NCHUNK:
            cps[nxt] = issue(nxt)
        cps[ci].wait()
        vtot = chunk_acc(ci, vtot)

    out_v[...] = vtot
    pltpu.sync_copy(out_v, out_hbm.at[wid])


@jax.jit
def kernel(x, labels, centers):
    centers_r = centers.reshape(NCLS, ROW)
    labels_i = labels.astype(jnp.int32)
    mesh = plsc.VectorSubcoreMesh(core_axis_name="c", subcore_axis_name="s")
    run = functools.partial(
        pl.kernel,
        mesh=mesh,
        out_type=jax.ShapeDtypeStruct((NWORK, LANES), jnp.float32),
        scratch_types=[
            pltpu.VMEM((SPW,), jnp.int32),
            pltpu.VMEM((SPW, FEAT), jnp.float32),
            pltpu.VMEM((NBUF, CHUNK, ROW), jnp.float32),
            pltpu.VMEM((LANES,), jnp.float32),
            pltpu.SemaphoreType.DMA,
            pltpu.SemaphoreType.DMA,
            pltpu.SemaphoreType.DMA,
            pltpu.SemaphoreType.DMA,
        ],
    )(_worker_body)
    partials = run(x, labels_i, centers_r)
    return jnp.sum(partials) / BATCH


# speedup vs baseline: 3.2858x; 1.0333x over previous
import functools

import jax
import jax.numpy as jnp
from jax import lax
from jax.experimental import pallas as pl
from jax.experimental.pallas import tpu as pltpu
from jax.experimental.pallas import tpu_sc as plsc

BATCH = 4096
FEAT = 512
NCORES = 2
NSUB = 16
NWORK = NCORES * NSUB
SPW = BATCH // NWORK
CHUNK = 16
NCHUNK = SPW // CHUNK
NBUF = 3
LANES = 16


def _lane_allreduce_sum(v):
    lane = lax.iota(jnp.int32, LANES)
    dnums = lax.GatherDimensionNumbers(
        offset_dims=(), collapsed_slice_dims=(0,), start_index_map=(0,))
    for k in (8, 4, 2, 1):
        perm = lane ^ k
        v = v + lax.gather(v, perm[:, None], dnums, slice_sizes=(1,),
                           mode=lax.GatherScatterMode.PROMISE_IN_BOUNDS)
    return v


def _worker_body(x_hbm, labels_hbm, centers_hbm, out_hbm,
                 labels_v, x_v, c0_v, c1_v, out_v,
                 sem_x, sem_c0, sem_c1, sem_c2):
    wid = lax.axis_index("s") * NCORES + lax.axis_index("c")
    base = wid * SPW
    sems = (sem_c0, sem_c1, sem_c2)

    cp_x = pltpu.async_copy(x_hbm.at[pl.ds(base, SPW)], x_v, sem_x)
    pltpu.sync_copy(labels_hbm.at[pl.ds(base, SPW)], labels_v)

    def issue(ci):
        b = ci % NBUF
        lv = labels_v[pl.ds(ci * CHUNK, CHUNK)]
        idx0 = lv + lv
        cpa = pltpu.async_copy(centers_hbm.at[idx0], c0_v.at[b], sems[b])
        cpb = pltpu.async_copy(centers_hbm.at[idx0 + 1], c1_v.at[b], sems[b])
        return (cpa, cpb)

    cps = {}
    for ci in range(NBUF - 1):
        cps[ci] = issue(ci)
    cp_x.wait()

    def chunk_acc(ci, vtot):
        b = ci % NBUF

        def sample_acc(s, vt):
            acc0 = jnp.zeros((LANES,), jnp.float32)
            acc1 = jnp.zeros((LANES,), jnp.float32)
            for j in range(FEAT // LANES):
                xv = x_v[ci * CHUNK + s, pl.ds(j * LANES, LANES)]
                d0 = xv - c0_v[b, s, pl.ds(j * LANES, LANES)]
                d1 = xv - c1_v[b, s, pl.ds(j * LANES, LANES)]
                acc0 = acc0 + d0 * d0
                acc1 = acc1 + d1 * d1
            take0 = _lane_allreduce_sum(acc0 - acc1) < 0.0
            sel = jnp.where(take0, acc0, acc1)
            return vt + sel

        return lax.fori_loop(0, CHUNK, sample_acc, vtot)

    vtot = jnp.zeros((LANES,), jnp.float32)
    for ci in range(NCHUNK):
        nxt = ci + NBUF - 1
        if nxt < NCHUNK:
            cps[nxt] = issue(nxt)
        cps[ci][0].wait()
        cps[ci][1].wait()
        vtot = chunk_acc(ci, vtot)

    out_v[...] = vtot
    pltpu.sync_copy(out_v, out_hbm.at[wid])


@jax.jit
def kernel(x, labels, centers):
    labels_i = labels.astype(jnp.int32)
    mesh = plsc.VectorSubcoreMesh(core_axis_name="c", subcore_axis_name="s")
    run = functools.partial(
        pl.kernel,
        mesh=mesh,
        out_type=jax.ShapeDtypeStruct((NWORK, LANES), jnp.float32),
        scratch_types=[
            pltpu.VMEM((SPW,), jnp.int32),
            pltpu.VMEM((SPW, FEAT), jnp.float32),
            pltpu.VMEM((NBUF, CHUNK, FEAT), jnp.float32),
            pltpu.VMEM((NBUF, CHUNK, FEAT), jnp.float32),
            pltpu.VMEM((LANES,), jnp.float32),
            pltpu.SemaphoreType.DMA,
            pltpu.SemaphoreType.DMA,
            pltpu.SemaphoreType.DMA,
            pltpu.SemaphoreType.DMA,
        ],
    )(_worker_body)
    partials = run(x, labels_i, centers)
    return jnp.sum(partials) / BATCH
